# baseline (speedup 1.0000x reference)
import math

import jax
import jax.numpy as jnp
from jax import lax
from jax.experimental import pallas as pl
from jax.experimental.pallas import tpu as pltpu
from jax.experimental.pallas import tpu_sc as plsc

_D = 64
_SCALE = math.sqrt(_D)
_NC = 2
_NS = 16
_NW = _NC * _NS
_L = 16
_B = 4096
_T = 200
_RC = 128
_TR = _T // 8
_NCHUNK = _T


def _sc_body(idx_hbm, base_hbm, tab_hbm, out_hbm,
             idx_bufs, xbufs, ebufs, obufs,
             sems_i, sems_x, sems_e, sems_w):
    wid = lax.axis_index("s") * _NC + lax.axis_index("c")
    lane_iota = lax.iota(jnp.int32, _L)
    u_pats = [(lane_iota + d) & 15 for d in range(16)]
    w_pats = [(u >> 3) * 1024 + (u & 7) * 128 + lane_iota for u in u_pats]

    def idx_src(c):
        return idx_hbm.at[c // 8, wid, pl.ds((c % 8) * _RC, _RC)]

    def fire_idx(c, par):
        pltpu.async_copy(idx_src(c), idx_bufs[par], sems_i[par])

    def wait_idx(c, par):
        pltpu.make_async_copy(idx_src(c), idx_bufs[par], sems_i[par]).wait()

    def fire_gathers(par):
        pltpu.async_copy(base_hbm.at[idx_bufs[par]], xbufs[par], sems_x[par])
        pltpu.async_copy(tab_hbm.at[idx_bufs[par]], ebufs[par], sems_e[par])

    def wait_gathers(par):
        pltpu.make_async_copy(base_hbm.at[idx_bufs[par]], xbufs[par],
                              sems_x[par]).wait()
        pltpu.make_async_copy(tab_hbm.at[idx_bufs[par]], ebufs[par],
                              sems_e[par]).wait()

    def drain_writes(par):
        for _ in range(8):
            pltpu.make_async_copy(obufs[par].at[pl.ds(0, 1024)],
                                  out_hbm.at[0, 0, wid], sems_w[par]).wait()

    def do_chunk(c, par):
        @pl.when(c + 1 < _NCHUNK)
        def _():
            wait_idx(c + 1, 1 - par)
            fire_gathers(1 - par)

        wait_gathers(par)

        @pl.when(c + 2 < _NCHUNK)
        def _():
            fire_idx(c + 2, par)

        @pl.when(c >= 2)
        def _():
            drain_writes(par)

        xbuf, ebuf, obuf = xbufs[par], ebufs[par], obufs[par]

        @plsc.parallel_loop(0, 8, unroll=2)
        def _tile_loop(j):
            rows = lane_iota + j * _L
            for ci in range(4):
                oc = ci * _L
                base_w = (ci * 2) * 1024 + j * _L
                for d in range(16):
                    cols = u_pats[d] + oc
                    widx = w_pats[d] + base_w
                    xv = plsc.load_gather(xbuf, [rows, cols])
                    ev = plsc.load_gather(ebuf, [rows, cols])
                    plsc.store_scatter(obuf, [widx], xv + _SCALE * ev)

        @pl.loop(0, 8)
        def _write_loop(tc):
            pltpu.async_copy(obuf.at[pl.ds(tc * 1024, 1024)],
                             out_hbm.at[c, tc, wid], sems_w[par])

    pltpu.sync_copy(idx_src(0), idx_bufs[0])
    fire_gathers(0)
    fire_idx(1, 1)

    @pl.loop(0, _NCHUNK, step=2)
    def _chunk_loop(g):
        do_chunk(g, 0)
        do_chunk(g + 1, 1)

    drain_writes(0)
    drain_writes(1)


def _make_sc_kernel():
    mesh = plsc.VectorSubcoreMesh(
        core_axis_name="c", subcore_axis_name="s",
        num_cores=_NC, num_subcores=_NS)
    return pl.kernel(
        _sc_body,
        out_type=jax.ShapeDtypeStruct((_T, 8, _NW, 1024), jnp.float32),
        mesh=mesh,
        compiler_params=pltpu.CompilerParams(
            use_tc_tiling_on_sc=False, needs_layout_passes=False,
            disable_bounds_checks=True, disable_semaphore_checks=True),
        scratch_types=[
            [pltpu.VMEM((_RC,), jnp.int32) for _ in range(2)],
            [pltpu.VMEM((_RC, 128), jnp.float32) for _ in range(2)],
            [pltpu.VMEM((_RC, 128), jnp.float32) for _ in range(2)],
            [pltpu.VMEM((8 * 1024,), jnp.float32) for _ in range(2)],
            [pltpu.SemaphoreType.DMA for _ in range(2)],
            [pltpu.SemaphoreType.DMA for _ in range(2)],
            [pltpu.SemaphoreType.DMA for _ in range(2)],
            [pltpu.SemaphoreType.DMA for _ in range(2)],
        ],
    )


def kernel(tokens, base_table, table):
    t4 = (jnp.asarray(tokens, jnp.int32)
          .reshape(_NW, 128, _TR, 8)
          .transpose(2, 0, 3, 1)
          .reshape(_TR, _NW, 1024))
    b2 = jnp.pad(base_table, ((0, 0), (0, 64)))
    t2 = jnp.pad(table, ((0, 0), (0, 64)))
    out5 = _make_sc_kernel()(t4, b2, t2)
    return (out5.reshape(_T, 8, _NW, 8, 128)
            .transpose(2, 4, 0, 1, 3)
            .reshape(_B, _T, _D))
